# baseline (speedup 1.0000x reference)
import jax
import jax.numpy as jnp
import numpy as np
from jax import lax
from jax.experimental import pallas as pl
from jax.experimental.pallas import tpu as pltpu
from jax.experimental.pallas import tpu_sc as plsc

_B = 4
_S = 2048
_H = 768
_N = 16
_NSAMP = _N // _B
_NW = 32
_TPW = _S // _NW
_G = 8
_CTOK = 8
_NCHUNK = _TPW // _CTOK
_L = 16


_IDX_MAP = (np.arange(_S * _G).reshape(_S, _G) % _B * _S
            + np.arange(_S)[:, None]).reshape(-1).astype(np.int32)


def _body(idxp_hbm, rel_hbm, pe_hbm, wte_hbm, out_hbm,
          idx_v, rows_v, blend_v, rel_v, pe_v, gsem, wsem):
    c = lax.axis_index("c")
    s = lax.axis_index("s")
    wid = s * 2 + c

    ioff = pl.multiple_of(wid * (_TPW * _G), 8)
    pltpu.sync_copy(idxp_hbm.at[pl.ds(ioff, _TPW * _G)], idx_v)

    def _gather(j):
        hs = []
        for tt in range(_CTOK):
            idx = idx_v.at[pl.ds((j * _CTOK + tt) * _G, _G)]
            hs.append(pltpu.async_copy(wte_hbm.at[idx],
                                       rows_v.at[j % 2, tt], gsem))
        return hs

    gh = _gather(0)

    @pl.when(wid < 2)
    def _():
        pltpu.sync_copy(rel_hbm, rel_v)
        pltpu.sync_copy(pe_hbm, pe_v)
        for ss in range(_G):
            r = rel_v[pl.ds((wid * _G + ss) * _L, _L)]
            one_m_r = 1.0 - r
            row = blend_v.at[ss]
            for h in range(_H // _L):
                p0 = pe_v[pl.ds(h * _L, _L)]
                p1 = pe_v[pl.ds(_H + h * _L, _L)]
                row[pl.ds(h * _L, _L)] = one_m_r * p0 + r * p1
        pltpu.sync_copy(blend_v, out_hbm.at[0, wid])

    wh = []
    for j in range(_NCHUNK):
        for h in gh:
            h.wait()
        if j >= 1:
            for h in wh[j - 1]:
                h.wait()
        if j + 1 < _NCHUNK:
            gh = _gather(j + 1)
        t0 = pl.multiple_of(1 + wid * _TPW + j * _CTOK, 1)
        buf = rows_v.at[j % 2]
        wh.append([
            pltpu.async_copy(buf, out_hbm.at[pl.ds(t0, _CTOK), k], wsem)
            for k in range(2)
        ])
    for h in wh[_NCHUNK - 1]:
        h.wait()


_sc_call = pl.kernel(
    _body,
    out_type=jax.ShapeDtypeStruct((_S + 1, 2, _G, _H), jnp.float32),
    mesh=plsc.VectorSubcoreMesh(core_axis_name="c", subcore_axis_name="s"),
    scratch_types=[
        pltpu.VMEM((_TPW * _G,), jnp.int32),
        pltpu.VMEM((2, _CTOK, _G, _H), jnp.float32),
        pltpu.VMEM((_G, _H), jnp.float32),
        pltpu.VMEM((_N * _L,), jnp.float32),
        pltpu.VMEM((2 * _H,), jnp.float32),
        pltpu.SemaphoreType.DMA,
        pltpu.SemaphoreType.DMA,
    ],
)


def kernel(input_ids, relevance, wte, prompt_embeds):
    ids = input_ids.astype(jnp.int32)
    idxp = ids.reshape(-1)[_IDX_MAP]
    rel = jnp.repeat(relevance.astype(jnp.float32), _L)
    pe = prompt_embeds.reshape(-1)
    out = _sc_call(idxp, rel, pe, wte)
    return out.reshape(_S + 1, _N, _H).transpose(1, 0, 2)


# speedup vs baseline: 3.3289x; 3.3289x over previous
import jax
import jax.numpy as jnp
from jax import lax
from jax.experimental import pallas as pl
from jax.experimental.pallas import tpu as pltpu
from jax.experimental.pallas import tpu_sc as plsc

_B = 4
_S = 2048
_H = 768
_N = 16
_NSAMP = _N // _B
_NW = 32
_TPW = _S // _NW
_G = 8
_CTOK = 8
_NCHUNK = _TPW // _CTOK
_L = 16


def _body(idxp_hbm, rel_hbm, pe_hbm, wte_hbm, out_hbm,
          idx_v, rows_v, blend_v, rel_v, pe_v, gsem, wsem):
    c = lax.axis_index("c")
    s = lax.axis_index("s")
    wid = s * 2 + c

    ioff = pl.multiple_of(wid * (_TPW * _G), 8)
    pltpu.sync_copy(idxp_hbm.at[pl.ds(ioff, _TPW * _G)], idx_v)

    def _gather(j):
        hs = []
        for tt in range(_CTOK):
            idx = idx_v.at[pl.ds((j * _CTOK + tt) * _G, _G)]
            hs.append(pltpu.async_copy(wte_hbm.at[idx],
                                       rows_v.at[j % 2, tt], gsem))
        return hs

    gh = _gather(0)

    @pl.when(wid < 2)
    def _():
        pltpu.sync_copy(rel_hbm, rel_v)
        pltpu.sync_copy(pe_hbm, pe_v)
        for ss in range(_G):
            r = rel_v[pl.ds((wid * _G + ss) * _L, _L)]
            one_m_r = 1.0 - r
            row = blend_v.at[ss]
            for h in range(_H // _L):
                p0 = pe_v[pl.ds(h * _L, _L)]
                p1 = pe_v[pl.ds(_H + h * _L, _L)]
                row[pl.ds(h * _L, _L)] = one_m_r * p0 + r * p1
        pltpu.sync_copy(blend_v, out_hbm.at[0, wid])

    wh = []
    for j in range(_NCHUNK):
        for h in gh:
            h.wait()
        if j >= 1:
            for h in wh[j - 1]:
                h.wait()
        if j + 1 < _NCHUNK:
            gh = _gather(j + 1)
        t0 = pl.multiple_of(1 + wid * _TPW + j * _CTOK, 1)
        buf = rows_v.at[j % 2]
        wh.append([
            pltpu.async_copy(buf, out_hbm.at[pl.ds(t0, _CTOK), k], wsem)
            for k in range(2)
        ])
    for h in wh[_NCHUNK - 1]:
        h.wait()


_sc_call = pl.kernel(
    _body,
    out_type=jax.ShapeDtypeStruct((_S + 1, 2, _G, _H), jnp.float32),
    mesh=plsc.VectorSubcoreMesh(core_axis_name="c", subcore_axis_name="s"),
    scratch_types=[
        pltpu.VMEM((_TPW * _G,), jnp.int32),
        pltpu.VMEM((2, _CTOK, _G, _H), jnp.float32),
        pltpu.VMEM((_G, _H), jnp.float32),
        pltpu.VMEM((_N * _L,), jnp.float32),
        pltpu.VMEM((2 * _H,), jnp.float32),
        pltpu.SemaphoreType.DMA,
        pltpu.SemaphoreType.DMA,
    ],
)


def kernel(input_ids, relevance, wte, prompt_embeds):
    ids = input_ids.astype(jnp.int32)
    idxp = jnp.broadcast_to(ids.T[:, None, :],
                            (_S, _G // _B, _B)).reshape(-1)
    rel = jnp.repeat(relevance.astype(jnp.float32), _L)
    pe = prompt_embeds.reshape(-1)
    out = _sc_call(idxp, rel, pe, wte)
    return out.reshape(_S + 1, _N, _H).transpose(1, 0, 2)
